# baseline (speedup 1.0000x reference)
import jax
import jax.numpy as jnp
from jax import lax
from jax.experimental import pallas as pl
from jax.experimental.pallas import tpu as pltpu
from jax.experimental.pallas import tpu_sc as plsc

_NUM_FIELDS = 26
_VOCAB = 100000
_EMBED = 32
_BATCH = 16384

_NC, _NS, _LANES = 2, 16, 16
_NW = _NC * _NS
_VB = (_VOCAB + 127) // 128
_NB = _NUM_FIELDS * _VB
_CAP = 128
_WL = 32768
_WPB = _WL // _NW
_HPAD = 640
_PROWS = _BATCH // 4
_TRASH = _PROWS
_ACCR = _PROWS + 8


def _sc_body(row0_hbm, col0_hbm, s0_hbm, n_hbm, hl_hbm, hb_hbm, w2_hbm,
             out_hbm, row0_v, col0_v, s0_v, n_v, blk_v, hl_v, hb_v,
             ctb_v, cidx_v, zb_v, acc_sh):
    cid = lax.axis_index("c")
    sid = lax.axis_index("s")
    wid = sid * _NC + cid

    pltpu.sync_copy(row0_hbm.at[wid], row0_v)
    pltpu.sync_copy(col0_hbm.at[wid], col0_v)
    pltpu.sync_copy(s0_hbm.at[wid], s0_v)
    pltpu.sync_copy(n_hbm.at[wid], n_v)

    zeros = jnp.zeros((_LANES,), jnp.float32)

    def zbody(j, carry):
        for g in range(128 // _LANES):
            zb_v[j, pl.ds(g * _LANES, _LANES)] = zeros
        return carry

    lax.fori_loop(0, _CAP, zbody, None, unroll=8)
    for q in range(_PROWS // _NS // _CAP):
        pltpu.sync_copy(zb_v, acc_sh.at[pl.ds(sid * (_PROWS // _NS)
                                              + q * _CAP, _CAP)])

    @pl.when(sid == 0)
    def _():
        pltpu.sync_copy(zb_v.at[pl.ds(0, 8)], acc_sh.at[pl.ds(_TRASH, 8)])

    plsc.subcore_barrier()

    iota = lax.iota(jnp.int32, _LANES)
    iota0 = iota
    iota1 = iota + _LANES

    def scalar_at(ref, k):
        r = k >> 7
        c = k & 127
        basec = c - lax.rem(c, _LANES)
        win = ref[r, pl.ds(basec, _LANES)]
        sel = iota == (c - basec)
        return jnp.sum(jnp.where(sel, win, 0))

    def ebody(k, carry):
        n = scalar_at(n_v, k)

        @pl.when(n > 0)
        def _():
            row0 = pl.multiple_of(scalar_at(row0_v, k), _EMBED)
            col0 = scalar_at(col0_v, k)
            s0 = scalar_at(s0_v, k)

            @pl.when(col0 < _VOCAB - 128)
            def _():
                c128 = pl.multiple_of(col0, 128)
                pltpu.sync_copy(w2_hbm.at[pl.ds(row0, _EMBED),
                                          pl.ds(c128, _CAP)], blk_v)

            @pl.when(col0 == _VOCAB - 32)
            def _():
                for c in range(_EMBED):
                    pltpu.sync_copy(
                        w2_hbm.at[row0 + c, pl.ds(_VOCAB - 32, 32)],
                        blk_v.at[c, pl.ds(0, 32)])

            a0 = pl.multiple_of(s0 - lax.rem(s0, 8), 8)
            off = s0 - a0
            pltpu.sync_copy(hl_hbm.at[pl.ds(a0, _CAP + 8)], hl_v)
            pltpu.sync_copy(hb_hbm.at[pl.ds(a0, _CAP + 8)], hb_v)
            trash = jnp.full((_LANES,), _TRASH, jnp.int32)
            for g in range(_CAP // _LANES):
                cidx_v[pl.ds(g * _LANES, _LANES)] = trash

            def hbody(h, carry2):
                pos = lax.broadcast(off + h, (_LANES,))
                lane = plsc.load_gather(hl_v, [pos])
                dest = plsc.load_gather(hb_v, [pos])
                v0 = plsc.load_gather(blk_v, [iota0, lane])
                v1 = plsc.load_gather(blk_v, [iota1, lane])
                for g in range(128 // _LANES):
                    ctb_v[h, pl.ds(g * _LANES, _LANES)] = zeros
                dq = (dest & 3) << 5
                plsc.store_scatter(ctb_v.at[h], [dq + iota0], v0)
                plsc.store_scatter(ctb_v.at[h], [dq + iota1], v1)
                hsp = lax.broadcast(h, (_LANES,))
                plsc.store_scatter(cidx_v, [hsp], dest >> 2,
                                   mask=iota == 0)
                return carry2

            lax.fori_loop(0, n, hbody, None)
            pltpu.sync_copy(ctb_v, acc_sh.at[cidx_v], add=True)

        return carry

    lax.fori_loop(0, _WPB, ebody, None)

    plsc.subcore_barrier()
    rows = _PROWS // _NS
    pltpu.sync_copy(acc_sh.at[pl.ds(sid * rows, rows)],
                    out_hbm.at[cid, pl.ds(sid * rows, rows)])


def kernel(x, W):
    b_iota = jnp.arange(_BATCH, dtype=jnp.int32)
    key = x.astype(jnp.int32) * _BATCH + b_iota[:, None]
    key_s = lax.sort(key.T, dimension=1)
    vs = key_s >> 14
    bs = key_s & (_BATCH - 1)
    blk = vs >> 7
    lane = vs - (blk << 7)
    hl = jnp.pad(lane.reshape(-1), (0, _HPAD))
    hb = jnp.pad(bs.reshape(-1), (0, _HPAD))

    fb = (jnp.arange(_NUM_FIELDS, dtype=jnp.int32)[:, None] * _VB
          + blk).reshape(-1)
    starts = jnp.searchsorted(
        fb, jnp.arange(_NB + 1, dtype=jnp.int32), side="left"
    ).astype(jnp.int32)
    nb = starts[1:] - starts[:-1]
    m = (nb + _CAP - 1) // _CAP
    ex = jnp.cumsum(m, dtype=jnp.int32) - m
    vb = jnp.repeat(jnp.arange(_NB, dtype=jnp.int32), m,
                    total_repeat_length=_WL)
    vidx = jnp.arange(_WL, dtype=jnp.int32) - ex[vb]
    wl_s0 = starts[vb] + vidx * _CAP
    wl_n = jnp.clip(nb[vb] - vidx * _CAP, 0, _CAP)
    fi = vb // _VB
    bv = vb - fi * _VB
    wl_row0 = fi * _EMBED
    wl_col0 = bv << 7

    deal = lambda a: a.reshape(_WPB, _NW).T.reshape(_NW, _WPB // 128, 128)

    w2 = W.transpose(0, 2, 1).reshape(_NUM_FIELDS * _EMBED, _VOCAB)

    mesh = plsc.VectorSubcoreMesh(
        core_axis_name="c", subcore_axis_name="s",
        num_cores=_NC, num_subcores=_NS,
    )
    f = pl.kernel(
        _sc_body,
        out_type=jax.ShapeDtypeStruct((_NC, _PROWS, 4 * _EMBED),
                                      jnp.float32),
        mesh=mesh,
        scratch_types=[
            pltpu.VMEM((_WPB // 128, 128), jnp.int32),
            pltpu.VMEM((_WPB // 128, 128), jnp.int32),
            pltpu.VMEM((_WPB // 128, 128), jnp.int32),
            pltpu.VMEM((_WPB // 128, 128), jnp.int32),
            pltpu.VMEM((_EMBED, _CAP), jnp.float32),
            pltpu.VMEM((_CAP + 8,), jnp.int32),
            pltpu.VMEM((_CAP + 8,), jnp.int32),
            pltpu.VMEM((_CAP, 4 * _EMBED), jnp.float32),
            pltpu.VMEM((_CAP,), jnp.int32),
            pltpu.VMEM((_CAP, 4 * _EMBED), jnp.float32),
            pltpu.VMEM_SHARED((_ACCR, 4 * _EMBED), jnp.float32),
        ],
        compiler_params=pltpu.CompilerParams(use_tc_tiling_on_sc=True,
                                             needs_layout_passes=False),
    )
    out2 = f(deal(wl_row0), deal(wl_col0), deal(wl_s0), deal(wl_n),
             hl, hb, w2)
    out2 = out2.reshape(_NC, _BATCH, _EMBED)
    return out2[0] + out2[1]


# speedup vs baseline: 1.1476x; 1.1476x over previous
import jax
import jax.numpy as jnp
from jax import lax
from jax.experimental import pallas as pl
from jax.experimental.pallas import tpu as pltpu
from jax.experimental.pallas import tpu_sc as plsc

_NUM_FIELDS = 26
_VOCAB = 100000
_EMBED = 32
_BATCH = 16384

_NC, _NS, _LANES = 2, 16, 16
_NW = _NC * _NS
_VB = (_VOCAB + 127) // 128
_NB = _NUM_FIELDS * _VB
_CAP = 128
_WL = 32768
_WPB = _WL // _NW
_HPAD = 640
_PROWS = _BATCH // 4
_TRASH = _PROWS
_ACCR = _PROWS + 8


def _sc_body(row0_hbm, col0_hbm, s0_hbm, n_hbm, hl_hbm, hb_hbm, w2_hbm,
             out_hbm, row0_v, col0_v, s0_v, n_v,
             blk0_v, blk1_v, hl0_v, hl1_v, hb0_v, hb1_v,
             ctb0_v, ctb1_v, cidx0_v, cidx1_v, zb_v, acc_sh,
             semt0, semt1, semh0, semh1, sems0, sems1):
    blkb = (blk0_v, blk1_v)
    hlb = (hl0_v, hl1_v)
    hbb = (hb0_v, hb1_v)
    ctbb = (ctb0_v, ctb1_v)
    cidxb = (cidx0_v, cidx1_v)
    semt = (semt0, semt1)
    semh = (semh0, semh1)
    sems = (sems0, sems1)
    cid = lax.axis_index("c")
    sid = lax.axis_index("s")
    wid = sid * _NC + cid

    pltpu.sync_copy(row0_hbm.at[wid], row0_v)
    pltpu.sync_copy(col0_hbm.at[wid], col0_v)
    pltpu.sync_copy(s0_hbm.at[wid], s0_v)
    pltpu.sync_copy(n_hbm.at[wid], n_v)

    zeros = jnp.zeros((_LANES,), jnp.float32)

    def zbody(j, carry):
        for g in range(128 // _LANES):
            zb_v[j, pl.ds(g * _LANES, _LANES)] = zeros
        return carry

    lax.fori_loop(0, _CAP, zbody, None, unroll=8)
    for q in range(_PROWS // _NS // _CAP):
        pltpu.sync_copy(zb_v, acc_sh.at[pl.ds(sid * (_PROWS // _NS)
                                              + q * _CAP, _CAP)])

    @pl.when(sid == 0)
    def _():
        pltpu.sync_copy(zb_v.at[pl.ds(0, 8)], acc_sh.at[pl.ds(_TRASH, 8)])

    plsc.subcore_barrier()

    iota = lax.iota(jnp.int32, _LANES)
    iota0 = iota
    iota1 = iota + _LANES

    def scalar_at(ref, k):
        r = k >> 7
        c = k & 127
        basec = c - lax.rem(c, _LANES)
        win = ref[r, pl.ds(basec, _LANES)]
        sel = iota == (c - basec)
        return jnp.sum(jnp.where(sel, win, 0))

    def extract(k):
        n = scalar_at(n_v, k)
        row0 = pl.multiple_of(scalar_at(row0_v, k), _EMBED)
        col0 = scalar_at(col0_v, k)
        s0 = scalar_at(s0_v, k)
        a0 = pl.multiple_of(s0 - lax.rem(s0, 8), 8)
        return n, row0, col0, a0, s0 - a0

    def fire(par, sc):
        n, row0, col0, a0, _ = sc
        row0 = pl.multiple_of(row0, _EMBED)
        a0 = pl.multiple_of(a0, 8)

        @pl.when(n > 0)
        def _():
            @pl.when(col0 < _VOCAB - 128)
            def _():
                c128 = pl.multiple_of(col0, 128)
                pltpu.async_copy(w2_hbm.at[pl.ds(row0, _EMBED),
                                           pl.ds(c128, _CAP)],
                                 blkb[par], semt[par])

            @pl.when(col0 == _VOCAB - 32)
            def _():
                for c in range(_EMBED):
                    pltpu.sync_copy(
                        w2_hbm.at[row0 + c, pl.ds(_VOCAB - 32, 32)],
                        blkb[par].at[c, pl.ds(0, 32)])

            pltpu.async_copy(hl_hbm.at[pl.ds(a0, _CAP + 8)],
                             hlb[par], semh[par])
            pltpu.async_copy(hb_hbm.at[pl.ds(a0, _CAP + 8)],
                             hbb[par], semh[par])

    def process(par, sc):
        n, row0, col0, a0, off = sc
        row0 = pl.multiple_of(row0, _EMBED)
        a0 = pl.multiple_of(a0, 8)

        @pl.when(n > 0)
        def _():
            @pl.when(col0 < _VOCAB - 128)
            def _():
                c128 = pl.multiple_of(col0, 128)
                pltpu.make_async_copy(
                    w2_hbm.at[pl.ds(row0, _EMBED), pl.ds(c128, _CAP)],
                    blkb[par], semt[par]).wait()

            pltpu.make_async_copy(hl_hbm.at[pl.ds(a0, _CAP + 8)],
                                  hlb[par], semh[par]).wait()
            pltpu.make_async_copy(hb_hbm.at[pl.ds(a0, _CAP + 8)],
                                  hbb[par], semh[par]).wait()

            trash = jnp.full((_LANES,), _TRASH, jnp.int32)
            for g in range(_CAP // _LANES):
                cidxb[par][pl.ds(g * _LANES, _LANES)] = trash

            def hbody(h, carry2):
                pos = lax.broadcast(off + h, (_LANES,))
                lane = plsc.load_gather(hlb[par], [pos])
                dest = plsc.load_gather(hbb[par], [pos])
                v0 = plsc.load_gather(blkb[par], [iota0, lane])
                v1 = plsc.load_gather(blkb[par], [iota1, lane])
                for g in range(128 // _LANES):
                    ctbb[par][h, pl.ds(g * _LANES, _LANES)] = zeros
                dq = (dest & 3) << 5
                plsc.store_scatter(ctbb[par].at[h], [dq + iota0], v0)
                plsc.store_scatter(ctbb[par].at[h], [dq + iota1], v1)
                hsp = lax.broadcast(h, (_LANES,))
                plsc.store_scatter(cidxb[par], [hsp], dest >> 2,
                                   mask=iota == 0)
                return carry2

            lax.fori_loop(0, n, hbody, None)
            pltpu.async_copy(ctbb[par], acc_sh.at[cidxb[par]],
                             sems[par], add=True)

    def drain_acc(par, n):
        @pl.when(n > 0)
        def _():
            pltpu.make_async_copy(ctbb[par],
                                  acc_sh.at[cidxb[par]],
                                  sems[par]).wait()

    sc0 = extract(0)
    fire(0, sc0)
    z32 = jnp.int32(0)

    def tbody(u, carry):
        sca, pn0, pn1 = carry
        drain_acc(0, pn0)
        scb = extract(2 * u + 1)
        fire(1, scb)
        process(0, sca)
        drain_acc(1, pn1)
        scn = lax.cond(2 * u + 2 < _WPB,
                       lambda: extract(2 * u + 2),
                       lambda: (z32, z32, z32, z32, z32))
        fire(0, scn)
        process(1, scb)
        return scn, sca[0], scb[0]

    _, pn0, pn1 = lax.fori_loop(0, _WPB // 2, tbody, (sc0, z32, z32))
    drain_acc(0, pn0)
    drain_acc(1, pn1)

    plsc.subcore_barrier()
    rows = _PROWS // _NS
    pltpu.sync_copy(acc_sh.at[pl.ds(sid * rows, rows)],
                    out_hbm.at[cid, pl.ds(sid * rows, rows)])


def kernel(x, W):
    b_iota = jnp.arange(_BATCH, dtype=jnp.int32)
    key = x.astype(jnp.int32) * _BATCH + b_iota[:, None]
    key_s = lax.sort(key.T, dimension=1)
    vs = key_s >> 14
    bs = key_s & (_BATCH - 1)
    blk = vs >> 7
    lane = vs - (blk << 7)
    hl = jnp.pad(lane.reshape(-1), (0, _HPAD))
    hb = jnp.pad(bs.reshape(-1), (0, _HPAD))

    fb = (jnp.arange(_NUM_FIELDS, dtype=jnp.int32)[:, None] * _VB
          + blk).reshape(-1)
    starts = jnp.searchsorted(
        fb, jnp.arange(_NB + 1, dtype=jnp.int32), side="left"
    ).astype(jnp.int32)
    nb = starts[1:] - starts[:-1]
    m = (nb + _CAP - 1) // _CAP
    ex = jnp.cumsum(m, dtype=jnp.int32) - m
    vb = jnp.repeat(jnp.arange(_NB, dtype=jnp.int32), m,
                    total_repeat_length=_WL)
    vidx = jnp.arange(_WL, dtype=jnp.int32) - ex[vb]
    wl_s0 = starts[vb] + vidx * _CAP
    wl_n = jnp.clip(nb[vb] - vidx * _CAP, 0, _CAP)
    fi = vb // _VB
    bv = vb - fi * _VB
    wl_row0 = fi * _EMBED
    wl_col0 = bv << 7

    deal = lambda a: a.reshape(_WPB, _NW).T.reshape(_NW, _WPB // 128, 128)

    w2 = W.transpose(0, 2, 1).reshape(_NUM_FIELDS * _EMBED, _VOCAB)

    mesh = plsc.VectorSubcoreMesh(
        core_axis_name="c", subcore_axis_name="s",
        num_cores=_NC, num_subcores=_NS,
    )
    f = pl.kernel(
        _sc_body,
        out_type=jax.ShapeDtypeStruct((_NC, _PROWS, 4 * _EMBED),
                                      jnp.float32),
        mesh=mesh,
        scratch_types=[
            pltpu.VMEM((_WPB // 128, 128), jnp.int32),
            pltpu.VMEM((_WPB // 128, 128), jnp.int32),
            pltpu.VMEM((_WPB // 128, 128), jnp.int32),
            pltpu.VMEM((_WPB // 128, 128), jnp.int32),
            pltpu.VMEM((_EMBED, _CAP), jnp.float32),
            pltpu.VMEM((_EMBED, _CAP), jnp.float32),
            pltpu.VMEM((_CAP + 8,), jnp.int32),
            pltpu.VMEM((_CAP + 8,), jnp.int32),
            pltpu.VMEM((_CAP + 8,), jnp.int32),
            pltpu.VMEM((_CAP + 8,), jnp.int32),
            pltpu.VMEM((_CAP, 4 * _EMBED), jnp.float32),
            pltpu.VMEM((_CAP, 4 * _EMBED), jnp.float32),
            pltpu.VMEM((_CAP,), jnp.int32),
            pltpu.VMEM((_CAP,), jnp.int32),
            pltpu.VMEM((_CAP, 4 * _EMBED), jnp.float32),
            pltpu.VMEM_SHARED((_ACCR, 4 * _EMBED), jnp.float32),
            pltpu.SemaphoreType.DMA,
            pltpu.SemaphoreType.DMA,
            pltpu.SemaphoreType.DMA,
            pltpu.SemaphoreType.DMA,
            pltpu.SemaphoreType.DMA,
            pltpu.SemaphoreType.DMA,
        ],
        compiler_params=pltpu.CompilerParams(use_tc_tiling_on_sc=True,
                                             needs_layout_passes=False),
    )
    out2 = f(deal(wl_row0), deal(wl_col0), deal(wl_s0), deal(wl_n),
             hl, hb, w2)
    out2 = out2.reshape(_NC, _BATCH, _EMBED)
    return out2[0] + out2[1]


# speedup vs baseline: 3.6632x; 3.1920x over previous
import jax
import jax.numpy as jnp
from jax import lax
from jax.experimental import pallas as pl
from jax.experimental.pallas import tpu as pltpu
from jax.experimental.pallas import tpu_sc as plsc

_NUM_FIELDS = 26
_VOCAB = 100000
_EMBED = 32
_BATCH = 16384

_NC, _NS, _LANES = 2, 16, 16
_NW = _NC * _NS
_BPW = _BATCH // _NW
_CHUNK = 128
_NCHUNK = _BPW // _CHUNK


def _sc_body(gx_hbm, w_hbm, out_hbm, idx_v, rows_v, acc_v, sem):
    wid = lax.axis_index("s") * _NC + lax.axis_index("c")
    base = wid * _BPW

    zeros = jnp.zeros((_LANES,), jnp.float32)

    def zbody(j, carry):
        acc_v[j, pl.ds(0, _LANES)] = zeros
        acc_v[j, pl.ds(_LANES, _LANES)] = zeros
        return carry

    lax.fori_loop(0, _BPW, zbody, None, unroll=8)

    def fbody(i, carry):
        pltpu.sync_copy(gx_hbm.at[i, pl.ds(wid * _NCHUNK, _NCHUNK)], idx_v)
        descs = [
            pltpu.async_copy(
                w_hbm.at[idx_v.at[c]],
                rows_v.at[pl.ds(c * _CHUNK, _CHUNK)],
                sem,
            )
            for c in range(_NCHUNK)
        ]
        for d in descs:
            d.wait()

        def abody(j, c2):
            plsc.addupdate(acc_v.at[j, pl.ds(0, _LANES)],
                           rows_v[j, pl.ds(0, _LANES)])
            plsc.addupdate(acc_v.at[j, pl.ds(_LANES, _LANES)],
                           rows_v[j, pl.ds(_LANES, _LANES)])
            return c2

        lax.fori_loop(0, _BPW, abody, None, unroll=8)
        return carry

    lax.fori_loop(0, _NUM_FIELDS, fbody, None)

    pltpu.sync_copy(acc_v, out_hbm.at[pl.ds(base, _BPW)])


def kernel(x, W):
    offs = jnp.arange(_NUM_FIELDS, dtype=jnp.int32) * _VOCAB
    gx = (x.T + offs[:, None]).reshape(_NUM_FIELDS, _BATCH // _CHUNK, _CHUNK)
    w_flat = W.reshape(_NUM_FIELDS * _VOCAB, _EMBED)
    mesh = plsc.VectorSubcoreMesh(
        core_axis_name="c", subcore_axis_name="s",
        num_cores=_NC, num_subcores=_NS,
    )
    f = pl.kernel(
        _sc_body,
        out_type=jax.ShapeDtypeStruct((_BATCH, _EMBED), jnp.float32),
        mesh=mesh,
        scratch_types=[
            pltpu.VMEM((_NCHUNK, _CHUNK), jnp.int32),
            pltpu.VMEM((_BPW, _EMBED), jnp.float32),
            pltpu.VMEM((_BPW, _EMBED), jnp.float32),
            pltpu.SemaphoreType.DMA,
        ],
        compiler_params=pltpu.CompilerParams(use_tc_tiling_on_sc=False),
    )
    return f(gx, w_flat)
